# baseline (speedup 1.0000x reference)
import functools

import jax
import jax.numpy as jnp
from jax.experimental import pallas as pl

_TOPK = 5
_BLOCK_N = 1024
_SUB = 4
_SUB_N = _BLOCK_N // _SUB


def _topk_kernel(f_ref, mt_ref, s_ref, i_ref):
    mt = mt_ref[...]
    mh = -0.5 * jnp.sum(mt * mt, axis=0, keepdims=True)
    k = mt.shape[1]

    subs = []
    for s in range(_SUB):
        f = f_ref[pl.ds(s * _SUB_N, _SUB_N), :]
        dot = jnp.dot(f, mt, preferred_element_type=jnp.float32)
        subs.append((f, dot + mh))

    iota_f = jax.lax.broadcasted_iota(
        jnp.int32, (_SUB_N, k), 1).astype(jnp.float32)
    for s, (f, g) in enumerate(subs):
        work = g
        top_s = []
        top_i = []
        for _ in range(_TOPK):
            cur_max = jnp.max(work, axis=1, keepdims=True)
            hit = work == cur_max
            cur_idx = jnp.min(jnp.where(hit, iota_f, float(k)),
                              axis=1, keepdims=True)
            top_s.append(cur_max)
            top_i.append(cur_idx)
            work = jnp.where(hit, -jnp.inf, work)
        xh = -0.5 * jnp.sum(f * f, axis=1, keepdims=True)
        rows = pl.ds(s * _SUB_N, _SUB_N)
        s_ref[rows, :] = jnp.concatenate(top_s, axis=1) + xh
        i_ref[rows, :] = jnp.concatenate(top_i, axis=1).astype(jnp.int32)


@functools.partial(jax.jit, static_argnames=())
def kernel(features, unit_means):
    n, d = features.shape
    k = unit_means.shape[0]
    mt = unit_means.T
    grid = (n // _BLOCK_N,)
    out_s, out_i = pl.pallas_call(
        _topk_kernel,
        grid=grid,
        in_specs=[
            pl.BlockSpec((_BLOCK_N, d), lambda i: (i, 0)),
            pl.BlockSpec((d, k), lambda i: (0, 0)),
        ],
        out_specs=[
            pl.BlockSpec((_BLOCK_N, _TOPK), lambda i: (i, 0)),
            pl.BlockSpec((_BLOCK_N, _TOPK), lambda i: (i, 0)),
        ],
        out_shape=[
            jax.ShapeDtypeStruct((n, _TOPK), jnp.float32),
            jax.ShapeDtypeStruct((n, _TOPK), jnp.int32),
        ],
    )(features, mt)
    return out_s, out_i


# speedup vs baseline: 1.0443x; 1.0443x over previous
import functools

import jax
import jax.numpy as jnp
from jax.experimental import pallas as pl

_TOPK = 5
_BLOCK_N = 512
_SUB = 2
_SUB_N = _BLOCK_N // _SUB


def _topk_kernel(f_ref, mt_ref, s_ref, i_ref):
    mt = mt_ref[...]
    mh = -0.5 * jnp.sum(mt * mt, axis=0, keepdims=True)
    k = mt.shape[1]

    subs = []
    for s in range(_SUB):
        f = f_ref[pl.ds(s * _SUB_N, _SUB_N), :]
        dot = jnp.dot(f, mt, preferred_element_type=jnp.float32)
        subs.append((f, dot + mh))

    iota_f = jax.lax.broadcasted_iota(
        jnp.int32, (_SUB_N, k), 1).astype(jnp.float32)
    works = [g for _, g in subs]
    top_s = [[] for _ in range(_SUB)]
    top_i = [[] for _ in range(_SUB)]
    for _ in range(_TOPK):
        for s in range(_SUB):
            work = works[s]
            cur_max = jnp.max(work, axis=1, keepdims=True)
            hit = work == cur_max
            cur_idx = jnp.min(jnp.where(hit, iota_f, float(k)),
                              axis=1, keepdims=True)
            top_s[s].append(cur_max)
            top_i[s].append(cur_idx)
            works[s] = jnp.where(hit, -jnp.inf, work)
    for s, (f, g) in enumerate(subs):
        xh = -0.5 * jnp.sum(f * f, axis=1, keepdims=True)
        rows = pl.ds(s * _SUB_N, _SUB_N)
        s_ref[rows, :] = jnp.concatenate(top_s[s], axis=1) + xh
        i_ref[rows, :] = jnp.concatenate(top_i[s], axis=1).astype(jnp.int32)


@functools.partial(jax.jit, static_argnames=())
def kernel(features, unit_means):
    n, d = features.shape
    k = unit_means.shape[0]
    mt = unit_means.T
    grid = (n // _BLOCK_N,)
    out_s, out_i = pl.pallas_call(
        _topk_kernel,
        grid=grid,
        in_specs=[
            pl.BlockSpec((_BLOCK_N, d), lambda i: (i, 0)),
            pl.BlockSpec((d, k), lambda i: (0, 0)),
        ],
        out_specs=[
            pl.BlockSpec((_BLOCK_N, _TOPK), lambda i: (i, 0)),
            pl.BlockSpec((_BLOCK_N, _TOPK), lambda i: (i, 0)),
        ],
        out_shape=[
            jax.ShapeDtypeStruct((n, _TOPK), jnp.float32),
            jax.ShapeDtypeStruct((n, _TOPK), jnp.int32),
        ],
    )(features, mt)
    return out_s, out_i
